# Initial kernel scaffold; baseline (speedup 1.0000x reference)
import functools

import jax
import jax.numpy as jnp
from jax import lax
from jax.experimental import pallas as pl
from jax.experimental.pallas import tpu as pltpu
from jax.experimental.pallas import tpu_sc as plsc

_N, _N1, _N2 = 50000, 10000, 2048
_D_IN, _D_H, _D_OUT = 128, 128, 41
_E1, _E2 = 320000, 65536

_NC, _NS = 2, 16
_NW = _NC * _NS


def _make_seg_sum(E, S, D, SUBLEN, SUB, NCH, ZR):
    EPW = NCH * SUB * SUBLEN
    assert EPW * _NW == E
    SP = S // _NS
    assert SP % ZR == 0 and SUBLEN % 8 == 0 and SUBLEN <= 128
    CPR = D // 16

    mesh = plsc.VectorSubcoreMesh(core_axis_name="c", subcore_axis_name="s")

    @functools.partial(
        pl.kernel,
        out_type=(
            jax.ShapeDtypeStruct((_NC, S, D), jnp.float32),
            jax.ShapeDtypeStruct((_NC, S, 16), jnp.float32),
        ),
        mesh=mesh,
        scratch_types=[
            pltpu.VMEM((SUB, SUBLEN), jnp.int32),
            pltpu.VMEM((SUB, SUBLEN), jnp.int32),
            pltpu.VMEM((SUB * SUBLEN, D), jnp.float32),
            pltpu.VMEM((SUBLEN, 16), jnp.float32),
            pltpu.VMEM((ZR, D), jnp.float32),
            pltpu.VMEM((SP, 16), jnp.float32),
            pltpu.VMEM_SHARED((S, D), jnp.float32),
            pltpu.VMEM_SHARED((S, 16), jnp.float32),
            pltpu.SemaphoreType.DMA,
        ],
    )
    def seg(src_hbm, dst_hbm, table_hbm, acc_out, cnt_out,
            idx_s, idx_d, rows, ones, zbuf, zcnt, acc_sh, cnt_sh, sem):
        c = lax.axis_index("c")
        s = lax.axis_index("s")
        wid = c * _NS + s

        def fill_ones(i, carry):
            ones[i, :] = jnp.full((16,), 1.0, jnp.float32)
            return carry
        lax.fori_loop(0, SUBLEN, fill_ones, 0)

        def fill_zb(i, carry):
            zbuf[i // CPR, pl.ds((i % CPR) * 16, 16)] = jnp.zeros((16,), jnp.float32)
            return carry
        lax.fori_loop(0, ZR * CPR, fill_zb, 0)

        def fill_zc(i, carry):
            zcnt[i, :] = jnp.zeros((16,), jnp.float32)
            return carry
        lax.fori_loop(0, SP, fill_zc, 0)

        row0 = s * SP
        for j in range(SP // ZR):
            pltpu.sync_copy(zbuf, acc_sh.at[pl.ds(row0 + j * ZR, ZR)])
        pltpu.sync_copy(zcnt, cnt_sh.at[pl.ds(row0, SP)])
        plsc.subcore_barrier()

        base = wid * EPW

        def chunk(k, carry):
            off = base + k * (SUB * SUBLEN)
            for j in range(SUB):
                pltpu.sync_copy(src_hbm.at[pl.ds(off + j * SUBLEN, SUBLEN)],
                                idx_s.at[j])
                pltpu.sync_copy(dst_hbm.at[pl.ds(off + j * SUBLEN, SUBLEN)],
                                idx_d.at[j])
            cps = [pltpu.async_copy(table_hbm.at[idx_s.at[j]],
                                    rows.at[pl.ds(j * SUBLEN, SUBLEN)], sem)
                   for j in range(SUB)]
            for cp in cps:
                cp.wait()
            for j in range(SUB):
                pltpu.sync_copy(rows.at[pl.ds(j * SUBLEN, SUBLEN)],
                                acc_sh.at[idx_d.at[j]], add=True)
                pltpu.sync_copy(ones, cnt_sh.at[idx_d.at[j]], add=True)
            return carry
        lax.fori_loop(0, NCH, chunk, 0)
        plsc.subcore_barrier()

        for j in range(SP // ZR):
            pltpu.sync_copy(acc_sh.at[pl.ds(row0 + j * ZR, ZR)],
                            acc_out.at[c, pl.ds(row0 + j * ZR, ZR)])
        pltpu.sync_copy(cnt_sh.at[pl.ds(row0, SP)],
                        cnt_out.at[c, pl.ds(row0, SP)])

    return seg


_seg1 = _make_seg_sum(_E1, _N1, _D_H, SUBLEN=80, SUB=5, NCH=25, ZR=125)
_seg2 = _make_seg_sum(_E2, _N2, 64, SUBLEN=128, SUB=4, NCH=4, ZR=128)


def _tc_mid_body(acc_ref, cnt_ref, x_ref, wl1_ref, bl1_ref, wr1_ref,
                 wl2p_ref, h_ref, z_ref):
    summed = acc_ref[0] + acc_ref[1]
    c = cnt_ref[0, :, 0] + cnt_ref[1, :, 0]
    mean = summed / jnp.maximum(c, 1.0)[:, None]
    dn = (((1,), (1,)), ((), ()))
    t = lax.dot_general(mean, wl1_ref[...], dn,
                        preferred_element_type=jnp.float32)
    t = t + bl1_ref[...]
    t = t + lax.dot_general(x_ref[...], wr1_ref[...], dn,
                            preferred_element_type=jnp.float32)
    h = jnp.maximum(t, 0.0)
    h_ref[...] = h
    z_ref[...] = lax.dot_general(h, wl2p_ref[...], dn,
                                 preferred_element_type=jnp.float32)


def _tc_mid(acc, cnt, x1, Wl1, bl1, Wr1, Wl2p):
    B = 1000
    G = _N1 // B
    return pl.pallas_call(
        _tc_mid_body,
        grid=(G,),
        in_specs=[
            pl.BlockSpec((2, B, _D_H), lambda i: (0, i, 0)),
            pl.BlockSpec((2, B, 16), lambda i: (0, i, 0)),
            pl.BlockSpec((B, _D_IN), lambda i: (i, 0)),
            pl.BlockSpec((_D_H, _D_IN), lambda i: (0, 0)),
            pl.BlockSpec((1, _D_H), lambda i: (0, 0)),
            pl.BlockSpec((_D_H, _D_IN), lambda i: (0, 0)),
            pl.BlockSpec((64, _D_H), lambda i: (0, 0)),
        ],
        out_specs=[
            pl.BlockSpec((B, _D_H), lambda i: (i, 0)),
            pl.BlockSpec((B, 64), lambda i: (i, 0)),
        ],
        out_shape=[
            jax.ShapeDtypeStruct((_N1, _D_H), jnp.float32),
            jax.ShapeDtypeStruct((_N1, 64), jnp.float32),
        ],
    )(acc, cnt, x1, Wl1, bl1, Wr1, Wl2p)


def _tc_final_body(acc_ref, cnt_ref, h_ref, wr2_ref, bl2_ref, out_ref):
    summed = acc_ref[0] + acc_ref[1]
    c = cnt_ref[0, :, 0] + cnt_ref[1, :, 0]
    mean = summed / jnp.maximum(c, 1.0)[:, None]
    dn = (((1,), (1,)), ((), ()))
    t = mean[:, :_D_OUT] + lax.dot_general(
        h_ref[...], wr2_ref[...], dn, preferred_element_type=jnp.float32)
    t = t + bl2_ref[...]
    m = jnp.max(t, axis=1, keepdims=True)
    e = t - m
    lse = jnp.log(jnp.sum(jnp.exp(e), axis=1, keepdims=True))
    out_ref[...] = e - lse


def _tc_final(acc, cnt, h2, Wr2, bl2):
    return pl.pallas_call(
        _tc_final_body,
        out_shape=jax.ShapeDtypeStruct((_N2, _D_OUT), jnp.float32),
    )(acc, cnt, h2, Wr2, bl2)


def kernel(x, edge_index1, edge_index2, Wl1, bl1, Wr1, Wl2, bl2, Wr2):
    x1 = x[:_N1]
    acc1, cnt1 = _seg1(edge_index1[0], edge_index1[1], x1)
    Wl2p = jnp.concatenate(
        [Wl2, jnp.zeros((64 - _D_OUT, _D_H), jnp.float32)], axis=0)
    h, z2 = _tc_mid(acc1, cnt1, x1, Wl1, bl1.reshape(1, -1), Wr1, Wl2p)
    acc2, cnt2 = _seg2(edge_index2[0], edge_index2[1], z2)
    return _tc_final(acc2, cnt2, h[:_N2], Wr2, bl2.reshape(1, -1))


# speedup vs baseline: 4.9964x; 4.9964x over previous
import functools

import jax
import jax.numpy as jnp
from jax import lax
from jax.experimental import pallas as pl
from jax.experimental.pallas import tpu as pltpu
from jax.experimental.pallas import tpu_sc as plsc

_N, _N1, _N2 = 50000, 10000, 2048
_D_IN, _D_H, _D_OUT = 128, 128, 41
_E1, _E2 = 320000, 65536

_NC, _NS = 2, 16
_NW = _NC * _NS
_D1 = _D_H + 16
_D2 = 80


def _make_seg_sum(E, S, D, SUBLEN, SUB, NCH, ZR):
    EPW = NCH * SUB * SUBLEN
    assert EPW * _NW == E
    SP = S // _NS
    assert SP % ZR == 0 and ZR <= SUB * SUBLEN
    assert SUBLEN % 8 == 0 and SUBLEN <= 128
    assert (D * 4) % 64 == 0

    mesh = plsc.VectorSubcoreMesh(core_axis_name="c", subcore_axis_name="s")

    @functools.partial(
        pl.kernel,
        out_type=jax.ShapeDtypeStruct((_NC, S, D), jnp.float32),
        mesh=mesh,
        compiler_params=pltpu.CompilerParams(use_tc_tiling_on_sc=False),
        scratch_types=[
            pltpu.VMEM((SUB, SUBLEN), jnp.int32),
            pltpu.VMEM((SUB, SUBLEN), jnp.int32),
            pltpu.VMEM((SUB * SUBLEN, D), jnp.float32),
            pltpu.VMEM_SHARED((S, D), jnp.float32),
            pltpu.SemaphoreType.DMA,
        ],
    )
    def seg(src_hbm, dst_hbm, table_hbm, acc_out,
            idx_s, idx_d, rows, acc_sh, sem):
        c = lax.axis_index("c")
        s = lax.axis_index("s")
        wid = c * _NS + s

        def fill_zb(i, carry):
            r = i // (D // 16)
            col = (i % (D // 16)) * 16
            rows[r, pl.ds(col, 16)] = jnp.zeros((16,), jnp.float32)
            return carry
        lax.fori_loop(0, ZR * (D // 16), fill_zb, 0)

        row0 = s * SP
        for j in range(SP // ZR):
            pltpu.sync_copy(rows.at[pl.ds(0, ZR)],
                            acc_sh.at[pl.ds(row0 + j * ZR, ZR)])
        plsc.subcore_barrier()

        base = wid * EPW

        def chunk(k, carry):
            off = base + k * (SUB * SUBLEN)
            for j in range(SUB):
                pltpu.sync_copy(src_hbm.at[pl.ds(off + j * SUBLEN, SUBLEN)],
                                idx_s.at[j])
                pltpu.sync_copy(dst_hbm.at[pl.ds(off + j * SUBLEN, SUBLEN)],
                                idx_d.at[j])
            cps = [pltpu.async_copy(table_hbm.at[idx_s.at[j]],
                                    rows.at[pl.ds(j * SUBLEN, SUBLEN)], sem)
                   for j in range(SUB)]
            for cp in cps:
                cp.wait()
            for j in range(SUB):
                pltpu.sync_copy(rows.at[pl.ds(j * SUBLEN, SUBLEN)],
                                acc_sh.at[idx_d.at[j]], add=True)
            return carry
        lax.fori_loop(0, NCH, chunk, 0)
        plsc.subcore_barrier()

        for j in range(SP // ZR):
            pltpu.sync_copy(acc_sh.at[pl.ds(row0 + j * ZR, ZR)],
                            acc_out.at[c, pl.ds(row0 + j * ZR, ZR)])

    return seg


_seg1 = _make_seg_sum(_E1, _N1, _D1, SUBLEN=40, SUB=5, NCH=50, ZR=125)
_seg2 = _make_seg_sum(_E2, _N2, _D2, SUBLEN=128, SUB=4, NCH=4, ZR=128)

_DN = (((1,), (1,)), ((), ()))


def _tc_pre_body(x_ref, wl1_ref, bl1_ref, wr1_ref, z_ref, r_ref):
    B = x_ref.shape[0]
    z = lax.dot_general(x_ref[...], wl1_ref[...], _DN,
                        preferred_element_type=jnp.float32)
    z_ref[...] = jnp.concatenate(
        [z, jnp.ones((B, 16), jnp.float32)], axis=1)
    r_ref[...] = bl1_ref[...] + lax.dot_general(
        x_ref[...], wr1_ref[...], _DN, preferred_element_type=jnp.float32)


def _tc_pre(x1, Wl1, bl1, Wr1):
    B = 1000
    return pl.pallas_call(
        _tc_pre_body,
        grid=(_N1 // B,),
        in_specs=[
            pl.BlockSpec((B, _D_IN), lambda i: (i, 0)),
            pl.BlockSpec((_D_H, _D_IN), lambda i: (0, 0)),
            pl.BlockSpec((1, _D_H), lambda i: (0, 0)),
            pl.BlockSpec((_D_H, _D_IN), lambda i: (0, 0)),
        ],
        out_specs=[
            pl.BlockSpec((B, _D1), lambda i: (i, 0)),
            pl.BlockSpec((B, _D_H), lambda i: (i, 0)),
        ],
        out_shape=[
            jax.ShapeDtypeStruct((_N1, _D1), jnp.float32),
            jax.ShapeDtypeStruct((_N1, _D_H), jnp.float32),
        ],
    )(x1, Wl1, bl1, Wr1)


def _tc_mid_body(acc_ref, r_ref, wl2_ref, h_ref, z_ref):
    B = r_ref.shape[0]
    summed = acc_ref[0] + acc_ref[1]
    cnt = summed[:, _D_H]
    mean = summed[:, :_D_H] / jnp.maximum(cnt, 1.0)[:, None]
    h = jnp.maximum(mean + r_ref[...], 0.0)
    h_ref[...] = h
    z = lax.dot_general(h, wl2_ref[...], _DN,
                        preferred_element_type=jnp.float32)
    z_ref[...] = jnp.concatenate(
        [z, jnp.zeros((B, 64 - _D_OUT), jnp.float32),
         jnp.ones((B, 16), jnp.float32)], axis=1)


def _tc_mid(acc, r1, Wl2):
    B = 1000
    return pl.pallas_call(
        _tc_mid_body,
        grid=(_N1 // B,),
        in_specs=[
            pl.BlockSpec((2, B, _D1), lambda i: (0, i, 0)),
            pl.BlockSpec((B, _D_H), lambda i: (i, 0)),
            pl.BlockSpec((_D_OUT, _D_H), lambda i: (0, 0)),
        ],
        out_specs=[
            pl.BlockSpec((B, _D_H), lambda i: (i, 0)),
            pl.BlockSpec((B, _D2), lambda i: (i, 0)),
        ],
        out_shape=[
            jax.ShapeDtypeStruct((_N1, _D_H), jnp.float32),
            jax.ShapeDtypeStruct((_N1, _D2), jnp.float32),
        ],
    )(acc, r1, Wl2)


def _tc_final_body(acc_ref, h_ref, wr2_ref, bl2_ref, out_ref):
    summed = acc_ref[0] + acc_ref[1]
    cnt = summed[:, 64]
    mean = summed[:, :_D_OUT] / jnp.maximum(cnt, 1.0)[:, None]
    t = mean + lax.dot_general(h_ref[...], wr2_ref[...], _DN,
                               preferred_element_type=jnp.float32)
    t = t + bl2_ref[...]
    m = jnp.max(t, axis=1, keepdims=True)
    e = t - m
    lse = jnp.log(jnp.sum(jnp.exp(e), axis=1, keepdims=True))
    out_ref[...] = e - lse


def _tc_final(acc, h2, Wr2, bl2):
    return pl.pallas_call(
        _tc_final_body,
        out_shape=jax.ShapeDtypeStruct((_N2, _D_OUT), jnp.float32),
    )(acc, h2, Wr2, bl2)


def kernel(x, edge_index1, edge_index2, Wl1, bl1, Wr1, Wl2, bl2, Wr2):
    x1 = x[:_N1]
    z1, r1 = _tc_pre(x1, Wl1, bl1.reshape(1, -1), Wr1)
    acc1 = _seg1(edge_index1[0], edge_index1[1], z1)
    h, z2 = _tc_mid(acc1, r1, Wl2)
    acc2 = _seg2(edge_index2[0], edge_index2[1], z2)
    return _tc_final(acc2, h[:_N2], Wr2, bl2.reshape(1, -1))
